# baseline (speedup 1.0000x reference)
import functools

import jax
import jax.numpy as jnp
from jax import lax
from jax.experimental import pallas as pl
from jax.experimental.pallas import tpu as pltpu
from jax.experimental.pallas import tpu_sc as plsc

_B, _H, _D = 4096, 200, 64
_HID, _OUT, _OPAD = 256, 2, 128

_NC, _NS, _L = 2, 16, 16
_NW = _NC * _NS
_NB = _B // _NW
_C0, _C1 = 128, 72
_NG = _D // _L


@functools.partial(
    pl.kernel,
    out_type=jax.ShapeDtypeStruct((_B, _D), jnp.float32),
    mesh=plsc.VectorSubcoreMesh(core_axis_name="c", subcore_axis_name="s"),
    compiler_params=pltpu.CompilerParams(use_tc_tiling_on_sc=False),
    name="sc_pool",
    scratch_types=[
        pltpu.VMEM((_NB * _H,), jnp.int32),
        pltpu.VMEM((_H, _D), jnp.float32),
        pltpu.VMEM((_H, _D), jnp.float32),
        pltpu.VMEM((_NB, _D), jnp.float32),
        pltpu.SemaphoreType.DMA,
        pltpu.SemaphoreType.DMA,
    ],
)
def _sc_pool(x_hbm, tab_hbm, out_hbm, idx_v, rows0, rows1, acc_v, sem0, sem1):
    wid = lax.axis_index("s") * _NC + lax.axis_index("c")
    base = wid * _NB
    pltpu.sync_copy(x_hbm.at[pl.ds(base * _H, _NB * _H)], idx_v)

    def start(b, rows, sem):
        off = b * _H
        pltpu.async_copy(
            tab_hbm.at[idx_v.at[pl.ds(off, _C0)]], rows.at[pl.ds(0, _C0)], sem)
        pltpu.async_copy(
            tab_hbm.at[idx_v.at[pl.ds(off + _C0, _C1)]],
            rows.at[pl.ds(_C0, _C1)], sem)

    def wait(rows, sem):
        pltpu.make_async_copy(tab_hbm.at[pl.ds(0, _H)], rows, sem).wait()

    def pool(rows, b):
        def body(jj, accs):
            j0 = jj * 8
            for dj in range(8):
                accs = tuple(
                    accs[g] + rows[j0 + dj, pl.ds(g * _L, _L)]
                    for g in range(_NG))
            return accs

        zero = jnp.zeros((_L,), jnp.float32)
        accs = lax.fori_loop(0, _H // 8, body, (zero,) * _NG)
        for g in range(_NG):
            acc_v[b, pl.ds(g * _L, _L)] = accs[g]

    start(0, rows0, sem0)

    def outer(i, carry):
        b = i * 2
        start(b + 1, rows1, sem1)
        wait(rows0, sem0)
        pool(rows0, b)

        @pl.when(b + 2 < _NB)
        def _():
            start(b + 2, rows0, sem0)

        wait(rows1, sem1)
        pool(rows1, b + 1)
        return carry

    lax.fori_loop(0, _NB // 2, outer, 0)
    pltpu.sync_copy(acc_v, out_hbm.at[pl.ds(base, _NB)])


def _mlp_body(e_ref, w1_ref, b1_ref, w2_ref, b2_ref, o_ref):
    h = jnp.dot(e_ref[...], w1_ref[...], preferred_element_type=jnp.float32)
    h = jnp.maximum(h + b1_ref[...], 0.0)
    o_ref[...] = (
        jnp.dot(h, w2_ref[...], preferred_element_type=jnp.float32)
        + b2_ref[...])


_BM = 512
_mlp = pl.pallas_call(
    _mlp_body,
    grid=(_B // _BM,),
    in_specs=[
        pl.BlockSpec((_BM, _D), lambda i: (i, 0)),
        pl.BlockSpec((_D, _HID), lambda i: (0, 0)),
        pl.BlockSpec((1, _HID), lambda i: (0, 0)),
        pl.BlockSpec((_HID, _OPAD), lambda i: (0, 0)),
        pl.BlockSpec((1, _OPAD), lambda i: (0, 0)),
    ],
    out_specs=pl.BlockSpec((_BM, _OPAD), lambda i: (i, 0)),
    out_shape=jax.ShapeDtypeStruct((_B, _OPAD), jnp.float32),
)


def kernel(x, emb_table, W1, b1, W2, b2):
    tabp = jnp.concatenate(
        [emb_table, jnp.zeros(emb_table.shape, emb_table.dtype)], axis=1
    ).reshape(-1, _D)
    pooled = _sc_pool((x * 2).reshape(-1), tabp)
    w2p = jnp.zeros((_HID, _OPAD), W2.dtype).at[:, :_OUT].set(W2)
    b2p = jnp.zeros((1, _OPAD), b2.dtype).at[0, :_OUT].set(b2)
    out = _mlp(pooled, W1, b1.reshape(1, _HID), w2p, b2p)
    return out[:, :_OUT]


# speedup vs baseline: 1.3688x; 1.3688x over previous
import functools

import jax
import jax.numpy as jnp
from jax import lax
from jax.experimental import pallas as pl
from jax.experimental.pallas import tpu as pltpu
from jax.experimental.pallas import tpu_sc as plsc

_B, _H, _D = 4096, 200, 64
_HID, _OUT, _OPAD = 256, 2, 128

_NC, _NS, _L = 2, 16, 16
_NW = _NC * _NS
_NB = _B // _NW
_C0, _C1 = 128, 72
_NG = _D // _L


@functools.partial(
    pl.kernel,
    out_type=jax.ShapeDtypeStruct((_B, _D), jnp.float32),
    mesh=plsc.VectorSubcoreMesh(core_axis_name="c", subcore_axis_name="s"),
    compiler_params=pltpu.CompilerParams(use_tc_tiling_on_sc=False),
    name="sc_pool",
    scratch_types=[
        pltpu.VMEM((_NB * _H,), jnp.int32),
        pltpu.VMEM((_H, _D), jnp.float32),
        pltpu.VMEM((_H, _D), jnp.float32),
        pltpu.VMEM((_NB, _D), jnp.float32),
        pltpu.SemaphoreType.DMA,
        pltpu.SemaphoreType.DMA,
    ],
)
def _sc_pool(x_hbm, tab_hbm, out_hbm, idx_v, rows0, rows1, acc_v, sem0, sem1):
    wid = lax.axis_index("s") * _NC + lax.axis_index("c")
    base = wid * _NB
    pltpu.sync_copy(x_hbm.at[pl.ds(base * _H, _NB * _H)], idx_v)

    def start(b, rows, sem):
        off = b * _H
        pltpu.async_copy(
            tab_hbm.at[idx_v.at[pl.ds(off, _C0)]], rows.at[pl.ds(0, _C0)], sem)
        pltpu.async_copy(
            tab_hbm.at[idx_v.at[pl.ds(off + _C0, _C1)]],
            rows.at[pl.ds(_C0, _C1)], sem)

    def wait(rows, sem):
        pltpu.make_async_copy(tab_hbm.at[pl.ds(0, _H)], rows, sem).wait()

    def pool(rows, b):
        def body(jj, accs):
            j0 = jj * 8
            for dj in range(8):
                accs = tuple(
                    accs[g] + rows[j0 + dj, pl.ds(g * _L, _L)]
                    for g in range(_NG))
            return accs

        zero = jnp.zeros((_L,), jnp.float32)
        accs = lax.fori_loop(0, _H // 8, body, (zero,) * _NG)
        for g in range(_NG):
            acc_v[b, pl.ds(g * _L, _L)] = accs[g]

    start(0, rows0, sem0)

    def outer(i, carry):
        b = i * 2
        start(b + 1, rows1, sem1)
        wait(rows0, sem0)
        pool(rows0, b)

        @pl.when(b + 2 < _NB)
        def _():
            start(b + 2, rows0, sem0)

        wait(rows1, sem1)
        pool(rows1, b + 1)
        return carry

    lax.fori_loop(0, _NB // 2, outer, 0)
    pltpu.sync_copy(acc_v, out_hbm.at[pl.ds(base, _NB)])


_V = 1000000
_VB = 4096


def _tr_body(t_ref, o_ref):
    blk = t_ref[...]
    o_ref[...] = jnp.concatenate(
        [blk.T, jnp.zeros((_VB, _D), jnp.float32)], axis=1)


_tr_pad = pl.pallas_call(
    _tr_body,
    grid=(pl.cdiv(_V, _VB),),
    in_specs=[pl.BlockSpec((_D, _VB), lambda i: (0, i))],
    out_specs=pl.BlockSpec((_VB, 2 * _D), lambda i: (i, 0)),
    out_shape=jax.ShapeDtypeStruct((_V, 2 * _D), jnp.float32),
)


def _mlp_body(e_ref, w1_ref, b1_ref, w2_ref, b2_ref, o_ref):
    h = jnp.dot(e_ref[...], w1_ref[...], preferred_element_type=jnp.float32)
    h = jnp.maximum(h + b1_ref[...], 0.0)
    o_ref[...] = (
        jnp.dot(h, w2_ref[...], preferred_element_type=jnp.float32)
        + b2_ref[...])


_BM = 512
_mlp = pl.pallas_call(
    _mlp_body,
    grid=(_B // _BM,),
    in_specs=[
        pl.BlockSpec((_BM, _D), lambda i: (i, 0)),
        pl.BlockSpec((_D, _HID), lambda i: (0, 0)),
        pl.BlockSpec((1, _HID), lambda i: (0, 0)),
        pl.BlockSpec((_HID, _OPAD), lambda i: (0, 0)),
        pl.BlockSpec((1, _OPAD), lambda i: (0, 0)),
    ],
    out_specs=pl.BlockSpec((_BM, _OPAD), lambda i: (i, 0)),
    out_shape=jax.ShapeDtypeStruct((_B, _OPAD), jnp.float32),
)


def kernel(x, emb_table, W1, b1, W2, b2):
    tabp = _tr_pad(emb_table.T).reshape(-1, _D)
    pooled = _sc_pool((x * 2).reshape(-1), tabp)
    w2p = jnp.zeros((_HID, _OPAD), W2.dtype).at[:, :_OUT].set(W2)
    b2p = jnp.zeros((1, _OPAD), b2.dtype).at[0, :_OUT].set(b2)
    out = _mlp(pooled, W1, b1.reshape(1, _HID), w2p, b2p)
    return out[:, :_OUT]


# speedup vs baseline: 1.6033x; 1.1714x over previous
import functools

import jax
import jax.numpy as jnp
from jax import lax
from jax.experimental import pallas as pl
from jax.experimental.pallas import tpu as pltpu
from jax.experimental.pallas import tpu_sc as plsc

_B, _H, _D = 4096, 200, 64
_HID, _OUT, _OPAD = 256, 2, 128

_NC, _NS, _L = 2, 16, 16
_NW = _NC * _NS
_NB = _B // _NW
_C0, _C1 = 128, 72
_NG = _D // _L


@functools.partial(
    pl.kernel,
    out_type=jax.ShapeDtypeStruct((_B, _D), jnp.float32),
    mesh=plsc.VectorSubcoreMesh(core_axis_name="c", subcore_axis_name="s"),
    compiler_params=pltpu.CompilerParams(use_tc_tiling_on_sc=False),
    name="sc_pool",
    scratch_types=[
        pltpu.VMEM((_NB * _H,), jnp.int32),
        pltpu.VMEM((_H, _D), jnp.float32),
        pltpu.VMEM((_H, _D), jnp.float32),
        pltpu.VMEM((_NB, _D), jnp.float32),
        pltpu.SemaphoreType.DMA,
        pltpu.SemaphoreType.DMA,
    ],
)
def _sc_pool(x_hbm, tab_hbm, out_hbm, idx_v, rows0, rows1, acc_v, sem0, sem1):
    wid = lax.axis_index("s") * _NC + lax.axis_index("c")
    base = wid * _NB
    pltpu.sync_copy(x_hbm.at[pl.ds(base * _H, _NB * _H)], idx_v)

    def start(b, rows, sem):
        off = b * _H
        pltpu.async_copy(
            tab_hbm.at[idx_v.at[pl.ds(off, _C0)]], rows.at[pl.ds(0, _C0)], sem)
        pltpu.async_copy(
            tab_hbm.at[idx_v.at[pl.ds(off + _C0, _C1)]],
            rows.at[pl.ds(_C0, _C1)], sem)

    def wait(rows, sem):
        pltpu.make_async_copy(tab_hbm.at[pl.ds(0, _H)], rows, sem).wait()

    def pool(rows, b):
        def body(jj, accs):
            j0 = jj * 8
            for dj in range(8):
                accs = tuple(
                    accs[g] + rows[j0 + dj, pl.ds(g * _L, _L)]
                    for g in range(_NG))
            return accs

        zero = jnp.zeros((_L,), jnp.float32)
        accs = lax.fori_loop(0, _H // 8, body, (zero,) * _NG)
        for g in range(_NG):
            acc_v[b, pl.ds(g * _L, _L)] = accs[g]

    start(0, rows0, sem0)

    def outer(i, carry):
        b = i * 2
        start(b + 1, rows1, sem1)
        wait(rows0, sem0)
        pool(rows0, b)

        @pl.when(b + 2 < _NB)
        def _():
            start(b + 2, rows0, sem0)

        wait(rows1, sem1)
        pool(rows1, b + 1)
        return carry

    lax.fori_loop(0, _NB // 2, outer, 0)
    pltpu.sync_copy(acc_v, out_hbm.at[pl.ds(base, _NB)])


_V = 1000000
_VB = 8192


def _tr_body(t_ref, o_ref):
    blk = t_ref[...]
    o_ref[...] = jnp.concatenate(
        [blk.T, jnp.zeros((_VB, _D), jnp.float32)], axis=1)


_tr_pad = pl.pallas_call(
    _tr_body,
    grid=(pl.cdiv(_V, _VB),),
    in_specs=[pl.BlockSpec((_D, _VB), lambda i: (0, i))],
    out_specs=pl.BlockSpec((_VB, 2 * _D), lambda i: (i, 0)),
    out_shape=jax.ShapeDtypeStruct((_V, 2 * _D), jnp.float32),
)


def _mlp_body(e_ref, w1_ref, b1_ref, w2_ref, b2_ref, o_ref):
    h = jnp.dot(e_ref[...], w1_ref[...], preferred_element_type=jnp.float32)
    h = jnp.maximum(h + b1_ref[...], 0.0)
    o_ref[...] = (
        jnp.dot(h, w2_ref[...], preferred_element_type=jnp.float32)
        + b2_ref[...])


_BM = 512
_mlp = pl.pallas_call(
    _mlp_body,
    grid=(_B // _BM,),
    in_specs=[
        pl.BlockSpec((_BM, _D), lambda i: (i, 0)),
        pl.BlockSpec((_D, _HID), lambda i: (0, 0)),
        pl.BlockSpec((1, _HID), lambda i: (0, 0)),
        pl.BlockSpec((_HID, _OPAD), lambda i: (0, 0)),
        pl.BlockSpec((1, _OPAD), lambda i: (0, 0)),
    ],
    out_specs=pl.BlockSpec((_BM, _OPAD), lambda i: (i, 0)),
    out_shape=jax.ShapeDtypeStruct((_B, _OPAD), jnp.float32),
)


def kernel(x, emb_table, W1, b1, W2, b2):
    tabp = _tr_pad(emb_table.T).reshape(-1, _D)
    pooled = _sc_pool((x * 2).reshape(-1), tabp)
    w2p = jnp.zeros((_HID, _OPAD), W2.dtype).at[:, :_OUT].set(W2)
    b2p = jnp.zeros((1, _OPAD), b2.dtype).at[0, :_OUT].set(b2)
    out = _mlp(pooled, W1, b1.reshape(1, _HID), w2p, b2p)
    return out[:, :_OUT]


# speedup vs baseline: 1.6834x; 1.0500x over previous
import functools

import jax
import jax.numpy as jnp
from jax import lax
from jax.experimental import pallas as pl
from jax.experimental.pallas import tpu as pltpu
from jax.experimental.pallas import tpu_sc as plsc

_B, _H, _D = 4096, 200, 64
_HID, _OUT, _OPAD = 256, 2, 128

_NC, _NS, _L = 2, 16, 16
_NW = _NC * _NS
_NB = _B // _NW
_C0, _C1 = 128, 72
_NG = _D // _L


@functools.partial(
    pl.kernel,
    out_type=jax.ShapeDtypeStruct((_B, _D), jnp.float32),
    mesh=plsc.VectorSubcoreMesh(core_axis_name="c", subcore_axis_name="s"),
    compiler_params=pltpu.CompilerParams(use_tc_tiling_on_sc=False),
    name="sc_pool",
    scratch_types=[
        pltpu.VMEM((_NB * _H,), jnp.int32),
        pltpu.VMEM((_H, _D), jnp.float32),
        pltpu.VMEM((_H, _D), jnp.float32),
        pltpu.VMEM((_NB, _D), jnp.float32),
        pltpu.SemaphoreType.DMA,
        pltpu.SemaphoreType.DMA,
    ],
)
def _sc_pool(x_hbm, tab_hbm, out_hbm, idx_v, rows0, rows1, acc_v, sem0, sem1):
    wid = lax.axis_index("s") * _NC + lax.axis_index("c")
    base = wid * _NB
    pltpu.sync_copy(x_hbm.at[pl.ds(base * _H, _NB * _H)], idx_v)

    def start(b, rows, sem):
        off = b * _H
        pltpu.async_copy(
            tab_hbm.at[idx_v.at[pl.ds(off, _C0)]], rows.at[pl.ds(0, _C0)], sem)
        pltpu.async_copy(
            tab_hbm.at[idx_v.at[pl.ds(off + _C0, _C1)]],
            rows.at[pl.ds(_C0, _C1)], sem)

    def wait(rows, sem):
        pltpu.make_async_copy(tab_hbm.at[pl.ds(0, _H)], rows, sem).wait()

    def pool(rows, b):
        def body(jj, accs):
            j0 = jj * 8
            for dj in range(8):
                accs = tuple(
                    accs[g] + rows[j0 + dj, pl.ds(g * _L, _L)]
                    for g in range(_NG))
            return accs

        zero = jnp.zeros((_L,), jnp.float32)
        accs = lax.fori_loop(0, _H // 8, body, (zero,) * _NG)
        for g in range(_NG):
            acc_v[b, pl.ds(g * _L, _L)] = accs[g]

    start(0, rows0, sem0)

    def outer(i, carry):
        b = i * 2
        start(b + 1, rows1, sem1)
        wait(rows0, sem0)
        pool(rows0, b)

        @pl.when(b + 2 < _NB)
        def _():
            start(b + 2, rows0, sem0)

        wait(rows1, sem1)
        pool(rows1, b + 1)
        return carry

    lax.fori_loop(0, _NB // 2, outer, 0)
    pltpu.sync_copy(acc_v, out_hbm.at[pl.ds(base, _NB)])


_V = 1000000
_VB = 16384


def _tr_body(t_ref, o_ref):
    blk = t_ref[...]
    o_ref[...] = jnp.concatenate(
        [blk.T, jnp.zeros((_VB, _D), jnp.float32)], axis=1)


_tr_pad = pl.pallas_call(
    _tr_body,
    grid=(pl.cdiv(_V, _VB),),
    in_specs=[pl.BlockSpec((_D, _VB), lambda i: (0, i))],
    out_specs=pl.BlockSpec((_VB, 2 * _D), lambda i: (i, 0)),
    out_shape=jax.ShapeDtypeStruct((_V, 2 * _D), jnp.float32),
)


def _mlp_body(e_ref, w1_ref, b1_ref, w2_ref, b2_ref, o_ref):
    h = jnp.dot(e_ref[...], w1_ref[...], preferred_element_type=jnp.float32)
    h = jnp.maximum(h + b1_ref[...], 0.0)
    o_ref[...] = (
        jnp.dot(h, w2_ref[...], preferred_element_type=jnp.float32)
        + b2_ref[...])


_BM = 512
_mlp = pl.pallas_call(
    _mlp_body,
    grid=(_B // _BM,),
    in_specs=[
        pl.BlockSpec((_BM, _D), lambda i: (i, 0)),
        pl.BlockSpec((_D, _HID), lambda i: (0, 0)),
        pl.BlockSpec((1, _HID), lambda i: (0, 0)),
        pl.BlockSpec((_HID, _OPAD), lambda i: (0, 0)),
        pl.BlockSpec((1, _OPAD), lambda i: (0, 0)),
    ],
    out_specs=pl.BlockSpec((_BM, _OPAD), lambda i: (i, 0)),
    out_shape=jax.ShapeDtypeStruct((_B, _OPAD), jnp.float32),
)


def kernel(x, emb_table, W1, b1, W2, b2):
    tabp = _tr_pad(emb_table.T).reshape(-1, _D)
    pooled = _sc_pool((x * 2).reshape(-1), tabp)
    w2p = jnp.zeros((_HID, _OPAD), W2.dtype).at[:, :_OUT].set(W2)
    b2p = jnp.zeros((1, _OPAD), b2.dtype).at[0, :_OUT].set(b2)
    out = _mlp(pooled, W1, b1.reshape(1, _HID), w2p, b2p)
    return out[:, :_OUT]


# speedup vs baseline: 1.7167x; 1.0198x over previous
import functools

import jax
import jax.numpy as jnp
from jax import lax
from jax.experimental import pallas as pl
from jax.experimental.pallas import tpu as pltpu
from jax.experimental.pallas import tpu_sc as plsc

_B, _H, _D = 4096, 200, 64
_HID, _OUT, _OPAD = 256, 2, 128

_NC, _NS, _L = 2, 16, 16
_NW = _NC * _NS
_NB = _B // _NW
_C0, _C1 = 128, 72
_NG = _D // _L


@functools.partial(
    pl.kernel,
    out_type=jax.ShapeDtypeStruct((_B, _D), jnp.float32),
    mesh=plsc.VectorSubcoreMesh(core_axis_name="c", subcore_axis_name="s"),
    compiler_params=pltpu.CompilerParams(use_tc_tiling_on_sc=False),
    name="sc_pool",
    scratch_types=[
        pltpu.VMEM((_NB * _H,), jnp.int32),
        pltpu.VMEM((_H, _D), jnp.float32),
        pltpu.VMEM((_H, _D), jnp.float32),
        pltpu.VMEM((_NB, _D), jnp.float32),
        pltpu.SemaphoreType.DMA,
        pltpu.SemaphoreType.DMA,
    ],
)
def _sc_pool(x_hbm, tab_hbm, out_hbm, idx_v, rows0, rows1, acc_v, sem0, sem1):
    wid = lax.axis_index("s") * _NC + lax.axis_index("c")
    base = wid * _NB
    pltpu.sync_copy(x_hbm.at[pl.ds(base * _H, _NB * _H)], idx_v)

    def start(b, rows, sem):
        off = b * _H
        pltpu.async_copy(
            tab_hbm.at[idx_v.at[pl.ds(off, _C0)]], rows.at[pl.ds(0, _C0)], sem)
        pltpu.async_copy(
            tab_hbm.at[idx_v.at[pl.ds(off + _C0, _C1)]],
            rows.at[pl.ds(_C0, _C1)], sem)

    def wait(rows, sem):
        pltpu.make_async_copy(tab_hbm.at[pl.ds(0, _H)], rows, sem).wait()

    def pool(rows, b):
        def body(jj, accs):
            j0 = jj * 8
            for dj in range(8):
                accs = tuple(
                    accs[g] + rows[j0 + dj, pl.ds(g * _L, _L)]
                    for g in range(_NG))
            return accs

        zero = jnp.zeros((_L,), jnp.float32)
        accs = lax.fori_loop(0, _H // 8, body, (zero,) * _NG)
        for g in range(_NG):
            acc_v[b, pl.ds(g * _L, _L)] = accs[g]

    start(0, rows0, sem0)

    def outer(i, carry):
        b = i * 2
        start(b + 1, rows1, sem1)
        wait(rows0, sem0)
        pool(rows0, b)

        @pl.when(b + 2 < _NB)
        def _():
            start(b + 2, rows0, sem0)

        wait(rows1, sem1)
        pool(rows1, b + 1)
        return carry

    lax.fori_loop(0, _NB // 2, outer, 0)
    pltpu.sync_copy(acc_v, out_hbm.at[pl.ds(base, _NB)])


_V = 1000000
_VB = 32768


def _tr_body(t_ref, o_ref):
    blk = t_ref[...]
    o_ref[...] = jnp.concatenate(
        [blk.T, jnp.zeros((_VB, _D), jnp.float32)], axis=1)


_tr_pad = pl.pallas_call(
    _tr_body,
    grid=(pl.cdiv(_V, _VB),),
    in_specs=[pl.BlockSpec((_D, _VB), lambda i: (0, i))],
    out_specs=pl.BlockSpec((_VB, 2 * _D), lambda i: (i, 0)),
    out_shape=jax.ShapeDtypeStruct((_V, 2 * _D), jnp.float32),
)


def _mlp_body(e_ref, w1_ref, b1_ref, w2_ref, b2_ref, o_ref):
    h = jnp.dot(e_ref[...], w1_ref[...], preferred_element_type=jnp.float32)
    h = jnp.maximum(h + b1_ref[...], 0.0)
    o_ref[...] = (
        jnp.dot(h, w2_ref[...], preferred_element_type=jnp.float32)
        + b2_ref[...])


_BM = 512
_mlp = pl.pallas_call(
    _mlp_body,
    grid=(_B // _BM,),
    in_specs=[
        pl.BlockSpec((_BM, _D), lambda i: (i, 0)),
        pl.BlockSpec((_D, _HID), lambda i: (0, 0)),
        pl.BlockSpec((1, _HID), lambda i: (0, 0)),
        pl.BlockSpec((_HID, _OPAD), lambda i: (0, 0)),
        pl.BlockSpec((1, _OPAD), lambda i: (0, 0)),
    ],
    out_specs=pl.BlockSpec((_BM, _OPAD), lambda i: (i, 0)),
    out_shape=jax.ShapeDtypeStruct((_B, _OPAD), jnp.float32),
)


def kernel(x, emb_table, W1, b1, W2, b2):
    tabp = _tr_pad(emb_table.T).reshape(-1, _D)
    pooled = _sc_pool((x * 2).reshape(-1), tabp)
    w2p = jnp.zeros((_HID, _OPAD), W2.dtype).at[:, :_OUT].set(W2)
    b2p = jnp.zeros((1, _OPAD), b2.dtype).at[0, :_OUT].set(b2)
    out = _mlp(pooled, W1, b1.reshape(1, _HID), w2p, b2p)
    return out[:, :_OUT]
